# baseline (speedup 1.0000x reference)
import jax
import jax.numpy as jnp
from jax.experimental import pallas as pl


_SYN = 4
_NBKG_PAD = 128


def _tc_body(idx_ref, w_ref, rob_ref, out_ref):
    bn = w_ref.shape[0]
    c_iota = jax.lax.broadcasted_iota(jnp.int32, (bn, _NBKG_PAD), 1)
    a = jnp.zeros((bn, _NBKG_PAD), dtype=jnp.float32)
    for s in range(_SYN):
        a = a + jnp.where(c_iota == idx_ref[:, 2 * s + 1 : 2 * s + 2],
                          w_ref[:, s : s + 1], 0.0)
    out_ref[0] = jax.lax.dot_general(
        rob_ref[...], a.astype(jnp.bfloat16),
        dimension_numbers=(((1,), (1,)), ((), ())),
        preferred_element_type=jnp.float32)


def _tc_spmm(rob_pad, idx_cat, w_cat, block_n=4096):
    t, n = rob_pad.shape[0], w_cat.shape[0]
    grid = (pl.cdiv(n, block_n),)
    return pl.pallas_call(
        _tc_body,
        grid=grid,
        in_specs=[
            pl.BlockSpec((block_n, 2 * _SYN), lambda i: (i, 0)),
            pl.BlockSpec((block_n, _SYN), lambda i: (i, 0)),
            pl.BlockSpec((t, _NBKG_PAD), lambda i: (0, 0)),
        ],
        out_specs=pl.BlockSpec((1, t, block_n), lambda i: (0, 0, i)),
        out_shape=jax.ShapeDtypeStruct((1, t, n), jnp.float32),
    )(idx_cat, w_cat, rob_pad)


def kernel(inp, rest_of_brain, w_v1, idx_v1, w_lm, idx_lm):
    t, nbkg = rest_of_brain.shape
    idx_cat = jnp.concatenate([idx_v1, idx_lm]).reshape(-1, 2 * _SYN)
    w_cat = jnp.concatenate([w_v1, w_lm]).reshape(-1, _SYN)
    rob_pad = jnp.pad(rest_of_brain, ((0, 0), (0, _NBKG_PAD - nbkg)))
    return _tc_spmm(rob_pad.astype(jnp.bfloat16), idx_cat, w_cat)


# speedup vs baseline: 1.1544x; 1.1544x over previous
import jax
import jax.numpy as jnp
from jax.experimental import pallas as pl


_SYN = 4
_NBKG_PAD = 128


def _tc_body(idx_ref, w_ref, rob_ref, out_ref):
    bn = w_ref.shape[0]
    idx_t = jnp.transpose(idx_ref[...])
    w_t = jnp.transpose(w_ref[...])
    c_iota = jax.lax.broadcasted_iota(jnp.int32, (_NBKG_PAD, bn), 0)
    at = jnp.zeros((_NBKG_PAD, bn), dtype=jnp.float32)
    for s in range(_SYN):
        at = at + jnp.where(c_iota == idx_t[2 * s + 1 : 2 * s + 2, :],
                            w_t[s : s + 1, :], 0.0)
    out_ref[0] = jnp.dot(rob_ref[...], at.astype(jnp.bfloat16),
                         preferred_element_type=jnp.float32)


def _tc_spmm(rob_pad, idx_cat, w_cat, block_n=4096):
    t, n = rob_pad.shape[0], w_cat.shape[0]
    grid = (pl.cdiv(n, block_n),)
    return pl.pallas_call(
        _tc_body,
        grid=grid,
        in_specs=[
            pl.BlockSpec((block_n, 2 * _SYN), lambda i: (i, 0)),
            pl.BlockSpec((block_n, _SYN), lambda i: (i, 0)),
            pl.BlockSpec((t, _NBKG_PAD), lambda i: (0, 0)),
        ],
        out_specs=pl.BlockSpec((1, t, block_n), lambda i: (0, 0, i)),
        out_shape=jax.ShapeDtypeStruct((1, t, n), jnp.float32),
    )(idx_cat, w_cat, rob_pad)


def kernel(inp, rest_of_brain, w_v1, idx_v1, w_lm, idx_lm):
    t, nbkg = rest_of_brain.shape
    idx_cat = jnp.concatenate([idx_v1, idx_lm]).reshape(-1, 2 * _SYN)
    w_cat = jnp.concatenate([w_v1, w_lm]).reshape(-1, _SYN)
    rob_pad = jnp.pad(rest_of_brain, ((0, 0), (0, _NBKG_PAD - nbkg)))
    return _tc_spmm(rob_pad.astype(jnp.bfloat16), idx_cat, w_cat)


# speedup vs baseline: 1.6657x; 1.4429x over previous
import jax
import jax.numpy as jnp
from jax.experimental import pallas as pl


_SYN = 4
_NBKG_PAD = 128
_KDIM = _SYN * _NBKG_PAD


def _tc_body(cols_ref, w_ref, rob4_ref, out_ref):
    bn = w_ref.shape[0]
    e = jnp.where(
        jax.lax.broadcasted_iota(jnp.int32, (_SYN, _KDIM), 1) // _NBKG_PAD
        == jax.lax.broadcasted_iota(jnp.int32, (_SYN, _KDIM), 0),
        1.0, 0.0)
    offs = (_NBKG_PAD * jax.lax.broadcasted_iota(
        jnp.int32, (1, _SYN), 1)).astype(jnp.float32)
    colsf = cols_ref[...].astype(jnp.float32) + offs
    cb = jnp.dot(colsf, e, preferred_element_type=jnp.float32)
    wb = jnp.dot(w_ref[...], e, preferred_element_type=jnp.float32)
    c_iota = jax.lax.broadcasted_iota(jnp.int32, (bn, _KDIM), 1)
    a4 = jnp.where(c_iota == cb.astype(jnp.int32), wb, 0.0)
    out_ref[0] = jax.lax.dot_general(
        rob4_ref[...], a4.astype(jnp.bfloat16),
        dimension_numbers=(((1,), (1,)), ((), ())),
        preferred_element_type=jnp.float32)


def _tc_spmm(rob4, cols4, w4, block_n=4096):
    t, n = rob4.shape[0], w4.shape[0]
    grid = (pl.cdiv(n, block_n),)
    return pl.pallas_call(
        _tc_body,
        grid=grid,
        in_specs=[
            pl.BlockSpec((block_n, _SYN), lambda i: (i, 0)),
            pl.BlockSpec((block_n, _SYN), lambda i: (i, 0)),
            pl.BlockSpec((t, _KDIM), lambda i: (0, 0)),
        ],
        out_specs=pl.BlockSpec((1, t, block_n), lambda i: (0, 0, i)),
        out_shape=jax.ShapeDtypeStruct((1, t, n), jnp.float32),
    )(cols4, w4, rob4)


def kernel(inp, rest_of_brain, w_v1, idx_v1, w_lm, idx_lm):
    t, nbkg = rest_of_brain.shape
    cols4 = jnp.concatenate([idx_v1[:, 1], idx_lm[:, 1]]).reshape(-1, _SYN)
    w4 = jnp.concatenate([w_v1, w_lm]).reshape(-1, _SYN)
    rob_pad = jnp.pad(rest_of_brain, ((0, 0), (0, _NBKG_PAD - nbkg)))
    rob4 = jnp.tile(rob_pad, (1, _SYN)).astype(jnp.bfloat16)
    return _tc_spmm(rob4, cols4, w4)


# speedup vs baseline: 1.6794x; 1.0082x over previous
import jax
import jax.numpy as jnp
from jax.experimental import pallas as pl


_SYN = 4
_NBKG_PAD = 128
_KDIM = _SYN * _NBKG_PAD


def _tc_body(cols_ref, w_ref, rob4_ref, out_ref):
    bn = w_ref.shape[0]
    e = jnp.where(
        jax.lax.broadcasted_iota(jnp.int32, (_SYN, _KDIM), 1) // _NBKG_PAD
        == jax.lax.broadcasted_iota(jnp.int32, (_SYN, _KDIM), 0),
        1.0, 0.0)
    colsf = cols_ref[...].astype(jnp.float32)
    cb = jnp.dot(colsf, e, preferred_element_type=jnp.float32)
    wb = jnp.dot(w_ref[...], e, preferred_element_type=jnp.float32)
    c_iota = jax.lax.broadcasted_iota(jnp.int32, (bn, _KDIM), 1)
    a4 = jnp.where((c_iota & (_NBKG_PAD - 1)) == cb.astype(jnp.int32),
                   wb, 0.0)
    out_ref[0] = jax.lax.dot_general(
        rob4_ref[...], a4.astype(jnp.bfloat16),
        dimension_numbers=(((1,), (1,)), ((), ())),
        preferred_element_type=jnp.float32)


def _tc_spmm(rob4, cols4, w4, block_n=4096):
    t, n = rob4.shape[0], w4.shape[0]
    grid = (pl.cdiv(n, block_n),)
    return pl.pallas_call(
        _tc_body,
        grid=grid,
        in_specs=[
            pl.BlockSpec((block_n, _SYN), lambda i: (i, 0)),
            pl.BlockSpec((block_n, _SYN), lambda i: (i, 0)),
            pl.BlockSpec((t, _KDIM), lambda i: (0, 0)),
        ],
        out_specs=pl.BlockSpec((1, t, block_n), lambda i: (0, 0, i)),
        out_shape=jax.ShapeDtypeStruct((1, t, n), jnp.float32),
    )(cols4, w4, rob4)


def kernel(inp, rest_of_brain, w_v1, idx_v1, w_lm, idx_lm):
    t, nbkg = rest_of_brain.shape
    cols4 = jnp.concatenate([idx_v1[:, 1], idx_lm[:, 1]]).reshape(-1, _SYN)
    w4 = jnp.concatenate([w_v1, w_lm]).reshape(-1, _SYN)
    rob_pad = jnp.pad(rest_of_brain, ((0, 0), (0, _NBKG_PAD - nbkg)))
    rob4 = jnp.tile(rob_pad, (1, _SYN)).astype(jnp.bfloat16)
    return _tc_spmm(rob4, cols4, w4)
